# baseline (speedup 1.0000x reference)
import functools
import jax
import jax.numpy as jnp
from jax import lax
from jax.experimental import pallas as pl
from jax.experimental.pallas import tpu as pltpu
from jax.experimental.pallas import tpu_sc as plsc

S, D, H, E, TOPK = 2048, 768, 768, 8, 2
NSLOT = S * TOPK
BLK = 256
NB = NSLOT // BLK + (E - 1)
NPAD = NB * BLK
CHUNK = 128
NCHUNK = NSLOT // CHUNK

NW = 32
K2_SLOTS = NSLOT // NW
K4_TOK = S // NW


def _router_body(x_ref, wr_ref, pos_ref, gm_ref, be_ref):
    xb = x_ref[...]
    logits = jnp.dot(xb, wr_ref[...], preferred_element_type=jnp.float32)
    lt = jnp.transpose(logits)

    Linc = (lax.broadcasted_iota(jnp.int32, (E, E), 0)
            >= lax.broadcasted_iota(jnp.int32, (E, E), 1)).astype(jnp.float32)
    m1 = jnp.max(lt, axis=0, keepdims=True)
    is1 = (lt == m1).astype(jnp.float32)
    first1 = is1 * (jnp.dot(Linc, is1, preferred_element_type=jnp.float32) == 1.0)
    masked = jnp.where(first1 > 0, -jnp.inf, lt)
    m2 = jnp.max(masked, axis=0, keepdims=True)
    is2 = (masked == m2).astype(jnp.float32)
    first2 = is2 * (jnp.dot(Linc, is2, preferred_element_type=jnp.float32) == 1.0)

    g1 = jax.nn.sigmoid(m1 - m2)
    g1c = jnp.transpose(g1)
    gm_ref[...] = jnp.concatenate(
        [jnp.broadcast_to(g1c, (S, 128)),
         jnp.broadcast_to(1.0 - g1c, (S, 128))], axis=1)

    ohT = jnp.concatenate([first1, first2], axis=1)
    cnt = jnp.sum(ohT, axis=1, keepdims=True)
    pcnt = jnp.floor((cnt + (BLK - 1)) * (1.0 / BLK)) * BLK
    SL = (lax.broadcasted_iota(jnp.int32, (E, E), 0)
          > lax.broadcasted_iota(jnp.int32, (E, E), 1)).astype(jnp.float32)
    off = jnp.dot(SL, pcnt, preferred_element_type=jnp.float32)

    US = (lax.broadcasted_iota(jnp.int32, (CHUNK, CHUNK), 0)
          < lax.broadcasted_iota(jnp.int32, (CHUNK, CHUNK), 1)
          ).astype(jnp.float32)
    run = jnp.zeros((E, 1), jnp.float32)
    for c in range(NCHUNK):
        ohc = ohT[:, c * CHUNK:(c + 1) * CHUNK]
        within = jnp.dot(ohc, US, preferred_element_type=jnp.float32)
        posc = jnp.sum((within + run + off) * ohc, axis=0)
        pos_ref[pl.ds(c * CHUNK, CHUNK)] = posc.astype(jnp.int32)
        run = run + jnp.sum(ohc, axis=1, keepdims=True)

    bs = (lax.broadcasted_iota(jnp.int32, (E, NB), 1) * BLK).astype(jnp.float32)
    bem = jnp.sum((bs >= off).astype(jnp.float32), axis=0, keepdims=True) - 1.0
    nrb = jnp.sum(pcnt, axis=0, keepdims=True) * (1.0 / BLK)
    be_ref[...] = jnp.concatenate([bem, nrb], axis=1).astype(jnp.int32)


@jax.jit
def _router(x2, Wr):
    return pl.pallas_call(
        _router_body,
        grid=(1,),
        in_specs=[
            pl.BlockSpec((S, D), lambda i: (0, 0)),
            pl.BlockSpec((D, E), lambda i: (0, 0)),
        ],
        out_specs=[
            pl.BlockSpec((NSLOT,), lambda i: (0,)),
            pl.BlockSpec((S, 256), lambda i: (0, 0)),
            pl.BlockSpec((1, NB + 1), lambda i: (0, 0)),
        ],
        out_shape=[
            jax.ShapeDtypeStruct((NSLOT,), jnp.int32),
            jax.ShapeDtypeStruct((S, 256), jnp.float32),
            jax.ShapeDtypeStruct((1, NB + 1), jnp.int32),
        ],
    )(x2, Wr)


@functools.lru_cache(maxsize=None)
def _make_dispatch():
    mesh = plsc.VectorSubcoreMesh(core_axis_name="c", subcore_axis_name="s")

    NCH = 4
    CH = K2_SLOTS // NCH
    DW = D + 128

    @functools.partial(
        pl.kernel,
        out_type=jax.ShapeDtypeStruct((NPAD, DW), jnp.float32),
        mesh=mesh,
        scratch_types=[
            [pltpu.VMEM((CH,), jnp.int32) for _ in range(NCH)],
            [pltpu.VMEM((CH, DW), jnp.float32) for _ in range(NCH)],
            pltpu.SemaphoreType.DMA,
        ],
    )
    def _dispatch(x_hbm, pos_hbm, gm_hbm, xs_hbm, idxs, rows, sem):
        wid = lax.axis_index("s") * 2 + lax.axis_index("c")
        base = wid * K2_SLOTS
        t0 = lax.rem(base, S)
        cps = []
        for j in range(NCH):
            pltpu.sync_copy(pos_hbm.at[pl.ds(base + j * CH, CH)], idxs[j])
            pltpu.sync_copy(x_hbm.at[pl.ds(t0 + j * CH, CH)],
                            rows[j].at[:, pl.ds(0, D)])

            @pl.when(wid < NW // 2)
            def _(j=j):
                pltpu.sync_copy(gm_hbm.at[pl.ds(t0 + j * CH, CH), pl.ds(0, 128)],
                                rows[j].at[:, pl.ds(D, 128)])

            @pl.when(wid >= NW // 2)
            def _(j=j):
                pltpu.sync_copy(gm_hbm.at[pl.ds(t0 + j * CH, CH), pl.ds(128, 128)],
                                rows[j].at[:, pl.ds(D, 128)])

            cps.append(pltpu.async_copy(rows[j], xs_hbm.at[idxs[j]], sem))
        for cp in cps:
            cp.wait()

    return _dispatch


def _gelu_exact(x):
    return 0.5 * x * (1.0 + lax.erf(x * jnp.float32(0.7071067811865476)))


def _mlp_body(be_ref, xs_ref, w1_ref, b1_ref, w2_ref, b2_ref, o_ref):
    i = pl.program_id(0)

    @pl.when(i < be_ref[0, NB])
    def _():
        e = be_ref[0, i]
        xb = xs_ref[:, pl.ds(0, D)]
        wcol = xs_ref[:, pl.ds(D, 1)]
        h = jnp.dot(xb, w1_ref[0], preferred_element_type=jnp.float32)
        h = _gelu_exact(h + b1_ref[pl.ds(e, 1)])
        o = jnp.dot(h, w2_ref[0], preferred_element_type=jnp.float32)
        o_ref[...] = (o + b2_ref[pl.ds(e, 1)]) * wcol


@jax.jit
def _grouped_mlp(be, xs, W1, b1, W2, b2):
    grid_spec = pltpu.PrefetchScalarGridSpec(
        num_scalar_prefetch=1,
        grid=(NB,),
        in_specs=[
            pl.BlockSpec((BLK, D + 128), lambda i, be: (i, 0)),
            pl.BlockSpec((1, D, H), lambda i, be: (be[0, i], 0, 0)),
            pl.BlockSpec((E, H), lambda i, be: (0, 0)),
            pl.BlockSpec((1, H, D), lambda i, be: (be[0, i], 0, 0)),
            pl.BlockSpec((E, D), lambda i, be: (0, 0)),
        ],
        out_specs=pl.BlockSpec((BLK, D), lambda i, be: (i, 0)),
    )
    return pl.pallas_call(
        _mlp_body,
        grid_spec=grid_spec,
        out_shape=jax.ShapeDtypeStruct((NPAD, D), jnp.float32),
        compiler_params=pltpu.CompilerParams(
            dimension_semantics=("arbitrary",),
        ),
    )(be, xs, W1, b1, W2, b2)


@functools.lru_cache(maxsize=None)
def _make_combine():
    mesh = plsc.VectorSubcoreMesh(core_axis_name="c", subcore_axis_name="s")

    @functools.partial(
        pl.kernel,
        out_type=jax.ShapeDtypeStruct((S, D), jnp.float32),
        mesh=mesh,
        scratch_types=[
            pltpu.VMEM((K4_TOK,), jnp.int32),
            pltpu.VMEM((K4_TOK,), jnp.int32),
            pltpu.VMEM((K4_TOK, D), jnp.float32),
            pltpu.VMEM((K4_TOK, D), jnp.float32),
            pltpu.SemaphoreType.DMA,
        ],
    )
    def _combine(so_hbm, pos_hbm, out_hbm, idx0_v, idx1_v, rows_v, rows1_v, sem):
        wid = lax.axis_index("s") * 2 + lax.axis_index("c")
        t0 = wid * K4_TOK
        pltpu.sync_copy(pos_hbm.at[pl.ds(t0, K4_TOK)], idx0_v)
        pltpu.sync_copy(pos_hbm.at[pl.ds(S + t0, K4_TOK)], idx1_v)
        cp0 = pltpu.async_copy(so_hbm.at[idx0_v], rows_v, sem)
        cp1 = pltpu.async_copy(so_hbm.at[idx1_v], rows1_v, sem)
        cp0.wait()
        cp1.wait()

        def per_token(t, _):
            for c in range(D // 16):
                sl = pl.ds(c * 16, 16)
                rows_v[t, sl] = rows_v[t, sl] + rows1_v[t, sl]
            return 0

        lax.fori_loop(0, K4_TOK, per_token, 0)
        pltpu.sync_copy(rows_v, out_hbm.at[pl.ds(t0, K4_TOK)])

    return _combine


def kernel(x, Wr, W1, b1, W2, b2):
    orig_shape = x.shape
    x2 = x.reshape(-1, x.shape[-1])
    pos, gm, be2 = _router(x2, Wr)
    xs = _make_dispatch()(x2, pos, gm)
    so = _grouped_mlp(be2, xs, W1, b1, W2, b2)
    out = _make_combine()(so, pos)
    return out.reshape(orig_shape)


# speedup vs baseline: 1.0627x; 1.0627x over previous
import functools
import jax
import jax.numpy as jnp
from jax import lax
from jax.experimental import pallas as pl
from jax.experimental.pallas import tpu as pltpu
from jax.experimental.pallas import tpu_sc as plsc

S, D, H, E, TOPK = 2048, 768, 768, 8, 2
NSLOT = S * TOPK
BLK = 256
NB = NSLOT // BLK + (E - 1)
NPAD = NB * BLK
CHUNK = 128
NCHUNK = NSLOT // CHUNK

NW = 32
K2_SLOTS = NSLOT // NW
K4_TOK = S // NW


def _router_body(x_ref, wr_ref, pos_ref, g1_ref, g2_ref, be_ref):
    xb = x_ref[...]
    logits = jnp.dot(xb, wr_ref[...], preferred_element_type=jnp.float32)
    lt = jnp.transpose(logits)

    Linc = (lax.broadcasted_iota(jnp.int32, (E, E), 0)
            >= lax.broadcasted_iota(jnp.int32, (E, E), 1)).astype(jnp.float32)
    m1 = jnp.max(lt, axis=0, keepdims=True)
    is1 = (lt == m1).astype(jnp.float32)
    first1 = is1 * (jnp.dot(Linc, is1, preferred_element_type=jnp.float32) == 1.0)
    masked = jnp.where(first1 > 0, -jnp.inf, lt)
    m2 = jnp.max(masked, axis=0, keepdims=True)
    is2 = (masked == m2).astype(jnp.float32)
    first2 = is2 * (jnp.dot(Linc, is2, preferred_element_type=jnp.float32) == 1.0)

    g1 = jax.nn.sigmoid(m1 - m2)
    g1_ref[...] = g1[0]
    g2_ref[...] = 1.0 - g1[0]

    ohT = jnp.concatenate([first1, first2], axis=1)
    cnt = jnp.sum(ohT, axis=1, keepdims=True)
    pcnt = jnp.floor((cnt + (BLK - 1)) * (1.0 / BLK)) * BLK
    SL = (lax.broadcasted_iota(jnp.int32, (E, E), 0)
          > lax.broadcasted_iota(jnp.int32, (E, E), 1)).astype(jnp.float32)
    off = jnp.dot(SL, pcnt, preferred_element_type=jnp.float32)

    US = (lax.broadcasted_iota(jnp.int32, (CHUNK, CHUNK), 0)
          < lax.broadcasted_iota(jnp.int32, (CHUNK, CHUNK), 1)
          ).astype(jnp.float32)
    run = jnp.zeros((E, 1), jnp.float32)
    for c in range(NCHUNK):
        ohc = ohT[:, c * CHUNK:(c + 1) * CHUNK]
        within = jnp.dot(ohc, US, preferred_element_type=jnp.float32)
        posc = jnp.sum((within + run + off) * ohc, axis=0)
        pos_ref[pl.ds(c * CHUNK, CHUNK)] = posc.astype(jnp.int32)
        run = run + jnp.sum(ohc, axis=1, keepdims=True)

    bs = (lax.broadcasted_iota(jnp.int32, (E, NB), 1) * BLK).astype(jnp.float32)
    bem = jnp.sum((bs >= off).astype(jnp.float32), axis=0, keepdims=True) - 1.0
    nrb = jnp.sum(pcnt, axis=0, keepdims=True) * (1.0 / BLK)
    be_ref[...] = jnp.concatenate([bem, nrb], axis=1).astype(jnp.int32)


@jax.jit
def _router(x2, Wr):
    return pl.pallas_call(
        _router_body,
        grid=(1,),
        in_specs=[
            pl.BlockSpec((S, D), lambda i: (0, 0)),
            pl.BlockSpec((D, E), lambda i: (0, 0)),
        ],
        out_specs=[
            pl.BlockSpec((NSLOT,), lambda i: (0,)),
            pl.BlockSpec((S,), lambda i: (0,)),
            pl.BlockSpec((S,), lambda i: (0,)),
            pl.BlockSpec((1, NB + 1), lambda i: (0, 0)),
        ],
        out_shape=[
            jax.ShapeDtypeStruct((NSLOT,), jnp.int32),
            jax.ShapeDtypeStruct((S,), jnp.float32),
            jax.ShapeDtypeStruct((S,), jnp.float32),
            jax.ShapeDtypeStruct((1, NB + 1), jnp.int32),
        ],
    )(x2, Wr)


@functools.lru_cache(maxsize=None)
def _make_dispatch():
    mesh = plsc.VectorSubcoreMesh(core_axis_name="c", subcore_axis_name="s")

    NCH = 2
    CH = K2_SLOTS // NCH

    @functools.partial(
        pl.kernel,
        out_type=jax.ShapeDtypeStruct((NPAD, D), jnp.float32),
        mesh=mesh,
        scratch_types=[
            [pltpu.VMEM((CH,), jnp.int32) for _ in range(NCH)],
            [pltpu.VMEM((CH, D), jnp.float32) for _ in range(NCH)],
            pltpu.SemaphoreType.DMA,
        ],
    )
    def _dispatch(x_hbm, pos_hbm, xs_hbm, idxs, rows, sem):
        wid = lax.axis_index("s") * 2 + lax.axis_index("c")
        base = wid * K2_SLOTS
        t0 = lax.rem(base, S)
        cps = []
        for j in range(NCH):
            pltpu.sync_copy(pos_hbm.at[pl.ds(base + j * CH, CH)], idxs[j])
            pltpu.sync_copy(x_hbm.at[pl.ds(t0 + j * CH, CH)], rows[j])
            cps.append(pltpu.async_copy(rows[j], xs_hbm.at[idxs[j]], sem))
        for cp in cps:
            cp.wait()

    return _dispatch


def _gelu_exact(x):
    return 0.5 * x * (1.0 + lax.erf(x * jnp.float32(0.7071067811865476)))


def _mlp_body(be_ref, xs_ref, w1_ref, b1_ref, w2_ref, b2_ref, o_ref):
    i = pl.program_id(0)

    @pl.when(i < be_ref[0, NB])
    def _():
        e = be_ref[0, i]
        h = jnp.dot(xs_ref[...], w1_ref[0], preferred_element_type=jnp.float32)
        h = _gelu_exact(h + b1_ref[pl.ds(e, 1)])
        o = jnp.dot(h, w2_ref[0], preferred_element_type=jnp.float32)
        o_ref[...] = o + b2_ref[pl.ds(e, 1)]


@jax.jit
def _grouped_mlp(be, xs, W1, b1, W2, b2):
    grid_spec = pltpu.PrefetchScalarGridSpec(
        num_scalar_prefetch=1,
        grid=(NB,),
        in_specs=[
            pl.BlockSpec((BLK, D), lambda i, be: (i, 0)),
            pl.BlockSpec((1, D, H), lambda i, be: (be[0, i], 0, 0)),
            pl.BlockSpec((E, H), lambda i, be: (0, 0)),
            pl.BlockSpec((1, H, D), lambda i, be: (be[0, i], 0, 0)),
            pl.BlockSpec((E, D), lambda i, be: (0, 0)),
        ],
        out_specs=pl.BlockSpec((BLK, D), lambda i, be: (i, 0)),
    )
    return pl.pallas_call(
        _mlp_body,
        grid_spec=grid_spec,
        out_shape=jax.ShapeDtypeStruct((NPAD, D), jnp.float32),
        compiler_params=pltpu.CompilerParams(
            dimension_semantics=("arbitrary",),
        ),
    )(be, xs, W1, b1, W2, b2)


@functools.lru_cache(maxsize=None)
def _make_combine():
    mesh = plsc.VectorSubcoreMesh(core_axis_name="c", subcore_axis_name="s")

    @functools.partial(
        pl.kernel,
        out_type=jax.ShapeDtypeStruct((S, D), jnp.float32),
        mesh=mesh,
        scratch_types=[
            pltpu.VMEM((K4_TOK,), jnp.int32),
            pltpu.VMEM((K4_TOK,), jnp.int32),
            pltpu.VMEM((K4_TOK,), jnp.float32),
            pltpu.VMEM((K4_TOK,), jnp.float32),
            pltpu.VMEM((K4_TOK, D), jnp.float32),
            pltpu.VMEM((K4_TOK, D), jnp.float32),
            pltpu.SemaphoreType.DMA,
        ],
    )
    def _combine(so_hbm, pos_hbm, g1_hbm, g2_hbm, out_hbm,
                 idx0_v, idx1_v, w0_v, w1_v, rows_v, rows1_v, sem):
        wid = lax.axis_index("s") * 2 + lax.axis_index("c")
        t0 = wid * K4_TOK
        pltpu.sync_copy(pos_hbm.at[pl.ds(t0, K4_TOK)], idx0_v)
        pltpu.sync_copy(pos_hbm.at[pl.ds(S + t0, K4_TOK)], idx1_v)
        pltpu.sync_copy(g1_hbm.at[pl.ds(t0, K4_TOK)], w0_v)
        pltpu.sync_copy(g2_hbm.at[pl.ds(t0, K4_TOK)], w1_v)
        cp0 = pltpu.async_copy(so_hbm.at[idx0_v], rows_v, sem)
        cp1 = pltpu.async_copy(so_hbm.at[idx1_v], rows1_v, sem)
        cp0.wait()
        cp1.wait()

        dnums = lax.GatherDimensionNumbers(
            offset_dims=(), collapsed_slice_dims=(0,), start_index_map=(0,))

        def per_token(t, _):
            base = (t // 16) * 16
            off = t - base

            def bcast(wv):
                return lax.gather(
                    wv[pl.ds(base, 16)], jnp.full((16, 1), off, jnp.int32),
                    dnums, slice_sizes=(1,),
                    mode=lax.GatherScatterMode.PROMISE_IN_BOUNDS)

            wv0 = bcast(w0_v)
            wv1 = bcast(w1_v)
            for c in range(D // 16):
                sl = pl.ds(c * 16, 16)
                rows_v[t, sl] = wv0 * rows_v[t, sl] + wv1 * rows1_v[t, sl]
            return 0

        lax.fori_loop(0, K4_TOK, per_token, 0)
        pltpu.sync_copy(rows_v, out_hbm.at[pl.ds(t0, K4_TOK)])

    return _combine


def kernel(x, Wr, W1, b1, W2, b2):
    orig_shape = x.shape
    x2 = x.reshape(-1, x.shape[-1])
    pos, g1, g2, be2 = _router(x2, Wr)
    xs = _make_dispatch()(x2, pos)
    so = _grouped_mlp(be2, xs, W1, b1, W2, b2)
    out = _make_combine()(so, pos, g1, g2)
    return out.reshape(orig_shape)


# speedup vs baseline: 1.0846x; 1.0207x over previous
import functools
import jax
import jax.numpy as jnp
from jax import lax
from jax.experimental import pallas as pl
from jax.experimental.pallas import tpu as pltpu
from jax.experimental.pallas import tpu_sc as plsc

S, D, H, E, TOPK = 2048, 768, 768, 8, 2
NSLOT = S * TOPK
BLK = 256
NB = NSLOT // BLK + (E - 1)
NPAD = NB * BLK
CHUNK = 128
NCHUNK = NSLOT // CHUNK

NW = 32
K2_SLOTS = NSLOT // NW
K4_TOK = S // NW


def _router_body(x_ref, wr_ref, pos_ref, g1_ref, g2_ref, be_ref):
    xb = x_ref[...]
    logits = jnp.dot(xb, wr_ref[...], preferred_element_type=jnp.float32)
    lt = jnp.transpose(logits)

    Linc = (lax.broadcasted_iota(jnp.int32, (E, E), 0)
            >= lax.broadcasted_iota(jnp.int32, (E, E), 1)).astype(jnp.float32)
    m1 = jnp.max(lt, axis=0, keepdims=True)
    is1 = (lt == m1).astype(jnp.float32)
    first1 = is1 * (jnp.dot(Linc, is1, preferred_element_type=jnp.float32) == 1.0)
    masked = jnp.where(first1 > 0, -jnp.inf, lt)
    m2 = jnp.max(masked, axis=0, keepdims=True)
    is2 = (masked == m2).astype(jnp.float32)
    first2 = is2 * (jnp.dot(Linc, is2, preferred_element_type=jnp.float32) == 1.0)

    g1 = jax.nn.sigmoid(m1 - m2)
    g1_ref[...] = g1[0]
    g2_ref[...] = 1.0 - g1[0]

    ohT = jnp.concatenate([first1, first2], axis=1)
    cnt = jnp.sum(ohT, axis=1, keepdims=True)
    pcnt = jnp.floor((cnt + (BLK - 1)) * (1.0 / BLK)) * BLK
    SL = (lax.broadcasted_iota(jnp.int32, (E, E), 0)
          > lax.broadcasted_iota(jnp.int32, (E, E), 1)).astype(jnp.float32)
    off = jnp.dot(SL, pcnt, preferred_element_type=jnp.float32)

    US = (lax.broadcasted_iota(jnp.int32, (CHUNK, CHUNK), 0)
          < lax.broadcasted_iota(jnp.int32, (CHUNK, CHUNK), 1)
          ).astype(jnp.float32)
    run = jnp.zeros((E, 1), jnp.float32)
    for c in range(NCHUNK):
        ohc = ohT[:, c * CHUNK:(c + 1) * CHUNK]
        within = jnp.dot(ohc, US, preferred_element_type=jnp.float32)
        posc = jnp.sum((within + run + off) * ohc, axis=0)
        pos_ref[pl.ds(c * CHUNK, CHUNK)] = posc.astype(jnp.int32)
        run = run + jnp.sum(ohc, axis=1, keepdims=True)

    bs = (lax.broadcasted_iota(jnp.int32, (E, NB), 1) * BLK).astype(jnp.float32)
    bem = jnp.sum((bs >= off).astype(jnp.float32), axis=0, keepdims=True) - 1.0
    nrb = jnp.sum(pcnt, axis=0, keepdims=True) * (1.0 / BLK)
    be_ref[...] = jnp.concatenate([bem, nrb], axis=1).astype(jnp.int32)


@jax.jit
def _router(x2, Wr):
    return pl.pallas_call(
        _router_body,
        grid=(1,),
        in_specs=[
            pl.BlockSpec((S, D), lambda i: (0, 0)),
            pl.BlockSpec((D, E), lambda i: (0, 0)),
        ],
        out_specs=[
            pl.BlockSpec((NSLOT,), lambda i: (0,)),
            pl.BlockSpec((S,), lambda i: (0,)),
            pl.BlockSpec((S,), lambda i: (0,)),
            pl.BlockSpec((1, NB + 1), lambda i: (0, 0)),
        ],
        out_shape=[
            jax.ShapeDtypeStruct((NSLOT,), jnp.int32),
            jax.ShapeDtypeStruct((S,), jnp.float32),
            jax.ShapeDtypeStruct((S,), jnp.float32),
            jax.ShapeDtypeStruct((1, NB + 1), jnp.int32),
        ],
    )(x2, Wr)


@functools.lru_cache(maxsize=None)
def _make_dispatch():
    mesh = plsc.VectorSubcoreMesh(core_axis_name="c", subcore_axis_name="s")

    NCH = 2
    CH = K2_SLOTS // NCH

    @functools.partial(
        pl.kernel,
        out_type=jax.ShapeDtypeStruct((NPAD, D), jnp.float32),
        mesh=mesh,
        scratch_types=[
            [pltpu.VMEM((CH,), jnp.int32) for _ in range(NCH)],
            [pltpu.VMEM((CH, D), jnp.float32) for _ in range(NCH)],
            pltpu.SemaphoreType.DMA,
            pltpu.SemaphoreType.DMA,
        ],
    )
    def _dispatch(x_hbm, pos_hbm, xs_hbm, idxs, rows, rsem, sem):
        wid = lax.axis_index("s") * 2 + lax.axis_index("c")
        base = wid * K2_SLOTS
        t0 = lax.rem(base, S)
        rcps = []
        for j in range(NCH):
            rcps.append(pltpu.async_copy(
                pos_hbm.at[pl.ds(base + j * CH, CH)], idxs[j], rsem))
            rcps.append(pltpu.async_copy(
                x_hbm.at[pl.ds(t0 + j * CH, CH)], rows[j], rsem))
        cps = []
        for j in range(NCH):
            rcps[2 * j].wait()
            rcps[2 * j + 1].wait()
            cps.append(pltpu.async_copy(rows[j], xs_hbm.at[idxs[j]], sem))
        for cp in cps:
            cp.wait()

    return _dispatch


def _gelu_exact(x):
    return 0.5 * x * (1.0 + lax.erf(x * jnp.float32(0.7071067811865476)))


def _mlp_body(be_ref, xs_ref, w1_ref, b1_ref, w2_ref, b2_ref, o_ref):
    i = pl.program_id(0)

    @pl.when(i < be_ref[0, NB])
    def _():
        e = be_ref[0, i]
        h = jnp.dot(xs_ref[...], w1_ref[0], preferred_element_type=jnp.float32)
        h = _gelu_exact(h + b1_ref[pl.ds(e, 1)])
        o = jnp.dot(h, w2_ref[0], preferred_element_type=jnp.float32)
        o_ref[...] = o + b2_ref[pl.ds(e, 1)]


@jax.jit
def _grouped_mlp(be, xs, W1, b1, W2, b2):
    grid_spec = pltpu.PrefetchScalarGridSpec(
        num_scalar_prefetch=1,
        grid=(NB,),
        in_specs=[
            pl.BlockSpec((BLK, D), lambda i, be: (i, 0)),
            pl.BlockSpec((1, D, H), lambda i, be: (be[0, i], 0, 0)),
            pl.BlockSpec((E, H), lambda i, be: (0, 0)),
            pl.BlockSpec((1, H, D), lambda i, be: (be[0, i], 0, 0)),
            pl.BlockSpec((E, D), lambda i, be: (0, 0)),
        ],
        out_specs=pl.BlockSpec((BLK, D), lambda i, be: (i, 0)),
    )
    return pl.pallas_call(
        _mlp_body,
        grid_spec=grid_spec,
        out_shape=jax.ShapeDtypeStruct((NPAD, D), jnp.float32),
        compiler_params=pltpu.CompilerParams(
            dimension_semantics=("arbitrary",),
        ),
    )(be, xs, W1, b1, W2, b2)


@functools.lru_cache(maxsize=None)
def _make_combine():
    mesh = plsc.VectorSubcoreMesh(core_axis_name="c", subcore_axis_name="s")

    @functools.partial(
        pl.kernel,
        out_type=jax.ShapeDtypeStruct((S, D), jnp.float32),
        mesh=mesh,
        scratch_types=[
            pltpu.VMEM((K4_TOK,), jnp.int32),
            pltpu.VMEM((K4_TOK,), jnp.int32),
            pltpu.VMEM((K4_TOK,), jnp.float32),
            pltpu.VMEM((K4_TOK,), jnp.float32),
            pltpu.VMEM((K4_TOK, D), jnp.float32),
            pltpu.VMEM((K4_TOK, D), jnp.float32),
            pltpu.SemaphoreType.DMA,
            pltpu.SemaphoreType.DMA,
        ],
    )
    def _combine(so_hbm, pos_hbm, g1_hbm, g2_hbm, out_hbm,
                 idx0_v, idx1_v, w0_v, w1_v, rows_v, rows1_v, rsem, sem):
        wid = lax.axis_index("s") * 2 + lax.axis_index("c")
        t0 = wid * K4_TOK
        i0 = pltpu.async_copy(pos_hbm.at[pl.ds(t0, K4_TOK)], idx0_v, rsem)
        i1 = pltpu.async_copy(pos_hbm.at[pl.ds(S + t0, K4_TOK)], idx1_v, rsem)
        g0 = pltpu.async_copy(g1_hbm.at[pl.ds(t0, K4_TOK)], w0_v, rsem)
        g1c = pltpu.async_copy(g2_hbm.at[pl.ds(t0, K4_TOK)], w1_v, rsem)
        i0.wait()
        cp0 = pltpu.async_copy(so_hbm.at[idx0_v], rows_v, sem)
        i1.wait()
        cp1 = pltpu.async_copy(so_hbm.at[idx1_v], rows1_v, sem)
        g0.wait()
        g1c.wait()
        cp0.wait()
        cp1.wait()

        dnums = lax.GatherDimensionNumbers(
            offset_dims=(), collapsed_slice_dims=(0,), start_index_map=(0,))

        def per_token(t, _):
            base = (t // 16) * 16
            off = t - base

            def bcast(wv):
                return lax.gather(
                    wv[pl.ds(base, 16)], jnp.full((16, 1), off, jnp.int32),
                    dnums, slice_sizes=(1,),
                    mode=lax.GatherScatterMode.PROMISE_IN_BOUNDS)

            wv0 = bcast(w0_v)
            wv1 = bcast(w1_v)
            for c in range(D // 16):
                sl = pl.ds(c * 16, 16)
                rows_v[t, sl] = wv0 * rows_v[t, sl] + wv1 * rows1_v[t, sl]
            return 0

        lax.fori_loop(0, K4_TOK, per_token, 0)
        pltpu.sync_copy(rows_v, out_hbm.at[pl.ds(t0, K4_TOK)])

    return _combine


def kernel(x, Wr, W1, b1, W2, b2):
    orig_shape = x.shape
    x2 = x.reshape(-1, x.shape[-1])
    pos, g1, g2, be2 = _router(x2, Wr)
    xs = _make_dispatch()(x2, pos)
    so = _grouped_mlp(be2, xs, W1, b1, W2, b2)
    out = _make_combine()(so, pos, g1, g2)
    return out.reshape(orig_shape)


# speedup vs baseline: 1.0906x; 1.0055x over previous
import functools
import jax
import jax.numpy as jnp
from jax import lax
from jax.experimental import pallas as pl
from jax.experimental.pallas import tpu as pltpu
from jax.experimental.pallas import tpu_sc as plsc

S, D, H, E, TOPK = 2048, 768, 768, 8, 2
NSLOT = S * TOPK
BLK = 256
NB = NSLOT // BLK + (E - 1)
NPAD = NB * BLK
CHUNK = 128
NCHUNK = NSLOT // CHUNK

NW = 32
K2_SLOTS = NSLOT // NW
K4_TOK = S // NW


def _router_body(x_ref, wr_ref, pos_ref, g1_ref, g2_ref, be_ref):
    xb = x_ref[...]
    logits = jnp.dot(xb, wr_ref[...], preferred_element_type=jnp.float32)
    lt = jnp.transpose(logits)

    Linc = (lax.broadcasted_iota(jnp.int32, (E, E), 0)
            >= lax.broadcasted_iota(jnp.int32, (E, E), 1)).astype(jnp.float32)
    m1 = jnp.max(lt, axis=0, keepdims=True)
    is1 = (lt == m1).astype(jnp.float32)
    first1 = is1 * (jnp.dot(Linc, is1, preferred_element_type=jnp.float32) == 1.0)
    masked = jnp.where(first1 > 0, -jnp.inf, lt)
    m2 = jnp.max(masked, axis=0, keepdims=True)
    is2 = (masked == m2).astype(jnp.float32)
    first2 = is2 * (jnp.dot(Linc, is2, preferred_element_type=jnp.float32) == 1.0)

    g1 = jax.nn.sigmoid(m1 - m2)
    g1_ref[...] = g1[0]
    g2_ref[...] = 1.0 - g1[0]

    ohT = jnp.concatenate([first1, first2], axis=1)
    cnt = jnp.sum(ohT, axis=1, keepdims=True)
    pcnt = jnp.floor((cnt + (BLK - 1)) * (1.0 / BLK)) * BLK
    SL = (lax.broadcasted_iota(jnp.int32, (E, E), 0)
          > lax.broadcasted_iota(jnp.int32, (E, E), 1)).astype(jnp.float32)
    off = jnp.dot(SL, pcnt, preferred_element_type=jnp.float32)

    US = (lax.broadcasted_iota(jnp.int32, (CHUNK, CHUNK), 0)
          < lax.broadcasted_iota(jnp.int32, (CHUNK, CHUNK), 1)
          ).astype(jnp.float32)
    run = jnp.zeros((E, 1), jnp.float32)
    for c in range(NCHUNK):
        ohc = ohT[:, c * CHUNK:(c + 1) * CHUNK]
        within = jnp.dot(ohc, US, preferred_element_type=jnp.float32)
        posc = jnp.sum((within + run + off) * ohc, axis=0)
        pos_ref[pl.ds(c * CHUNK, CHUNK)] = posc.astype(jnp.int32)
        run = run + jnp.sum(ohc, axis=1, keepdims=True)

    bs = (lax.broadcasted_iota(jnp.int32, (E, NB), 1) * BLK).astype(jnp.float32)
    bem = jnp.sum((bs >= off).astype(jnp.float32), axis=0, keepdims=True) - 1.0
    nrb = jnp.sum(pcnt, axis=0, keepdims=True) * (1.0 / BLK)
    be_ref[...] = jnp.concatenate([bem, nrb], axis=1).astype(jnp.int32)


@jax.jit
def _router(x2, Wr):
    return pl.pallas_call(
        _router_body,
        grid=(1,),
        in_specs=[
            pl.BlockSpec((S, D), lambda i: (0, 0)),
            pl.BlockSpec((D, E), lambda i: (0, 0)),
        ],
        out_specs=[
            pl.BlockSpec((NSLOT,), lambda i: (0,)),
            pl.BlockSpec((S,), lambda i: (0,)),
            pl.BlockSpec((S,), lambda i: (0,)),
            pl.BlockSpec((1, NB + 1), lambda i: (0, 0)),
        ],
        out_shape=[
            jax.ShapeDtypeStruct((NSLOT,), jnp.int32),
            jax.ShapeDtypeStruct((S,), jnp.float32),
            jax.ShapeDtypeStruct((S,), jnp.float32),
            jax.ShapeDtypeStruct((1, NB + 1), jnp.int32),
        ],
    )(x2, Wr)


@functools.lru_cache(maxsize=None)
def _make_dispatch():
    mesh = plsc.VectorSubcoreMesh(core_axis_name="c", subcore_axis_name="s")

    NCH = 4
    CH = K2_SLOTS // NCH

    @functools.partial(
        pl.kernel,
        out_type=jax.ShapeDtypeStruct((NPAD, D), jnp.float32),
        mesh=mesh,
        scratch_types=[
            [pltpu.VMEM((CH,), jnp.int32) for _ in range(NCH)],
            [pltpu.VMEM((CH, D), jnp.float32) for _ in range(NCH)],
            pltpu.SemaphoreType.DMA,
            pltpu.SemaphoreType.DMA,
        ],
    )
    def _dispatch(x_hbm, pos_hbm, xs_hbm, idxs, rows, rsem, sem):
        wid = lax.axis_index("s") * 2 + lax.axis_index("c")
        base = wid * K2_SLOTS
        t0 = lax.rem(base, S)
        rcps = []
        for j in range(NCH):
            rcps.append(pltpu.async_copy(
                pos_hbm.at[pl.ds(base + j * CH, CH)], idxs[j], rsem))
            rcps.append(pltpu.async_copy(
                x_hbm.at[pl.ds(t0 + j * CH, CH)], rows[j], rsem))
        cps = []
        for j in range(NCH):
            rcps[2 * j].wait()
            rcps[2 * j + 1].wait()
            cps.append(pltpu.async_copy(rows[j], xs_hbm.at[idxs[j]], sem))
        for cp in cps:
            cp.wait()

    return _dispatch


def _gelu_exact(x):
    return 0.5 * x * (1.0 + lax.erf(x * jnp.float32(0.7071067811865476)))


def _mlp_body(be_ref, xs_ref, w1_ref, b1_ref, w2_ref, b2_ref, o_ref):
    i = pl.program_id(0)

    @pl.when(i < be_ref[0, NB])
    def _():
        e = be_ref[0, i]
        h = jnp.dot(xs_ref[...], w1_ref[0], preferred_element_type=jnp.float32)
        h = _gelu_exact(h + b1_ref[pl.ds(e, 1)])
        o = jnp.dot(h, w2_ref[0], preferred_element_type=jnp.float32)
        o_ref[...] = o + b2_ref[pl.ds(e, 1)]


@jax.jit
def _grouped_mlp(be, xs, W1, b1, W2, b2):
    grid_spec = pltpu.PrefetchScalarGridSpec(
        num_scalar_prefetch=1,
        grid=(NB,),
        in_specs=[
            pl.BlockSpec((BLK, D), lambda i, be: (i, 0)),
            pl.BlockSpec((1, D, H), lambda i, be: (be[0, i], 0, 0)),
            pl.BlockSpec((E, H), lambda i, be: (0, 0)),
            pl.BlockSpec((1, H, D), lambda i, be: (be[0, i], 0, 0)),
            pl.BlockSpec((E, D), lambda i, be: (0, 0)),
        ],
        out_specs=pl.BlockSpec((BLK, D), lambda i, be: (i, 0)),
    )
    return pl.pallas_call(
        _mlp_body,
        grid_spec=grid_spec,
        out_shape=jax.ShapeDtypeStruct((NPAD, D), jnp.float32),
        compiler_params=pltpu.CompilerParams(
            dimension_semantics=("arbitrary",),
        ),
    )(be, xs, W1, b1, W2, b2)


@functools.lru_cache(maxsize=None)
def _make_combine():
    mesh = plsc.VectorSubcoreMesh(core_axis_name="c", subcore_axis_name="s")

    @functools.partial(
        pl.kernel,
        out_type=jax.ShapeDtypeStruct((S, D), jnp.float32),
        mesh=mesh,
        scratch_types=[
            pltpu.VMEM((K4_TOK,), jnp.int32),
            pltpu.VMEM((K4_TOK,), jnp.int32),
            pltpu.VMEM((K4_TOK,), jnp.float32),
            pltpu.VMEM((K4_TOK,), jnp.float32),
            pltpu.VMEM((K4_TOK, D), jnp.float32),
            pltpu.VMEM((K4_TOK, D), jnp.float32),
            pltpu.SemaphoreType.DMA,
            pltpu.SemaphoreType.DMA,
        ],
    )
    def _combine(so_hbm, pos_hbm, g1_hbm, g2_hbm, out_hbm,
                 idx0_v, idx1_v, w0_v, w1_v, rows_v, rows1_v, rsem, sem):
        wid = lax.axis_index("s") * 2 + lax.axis_index("c")
        t0 = wid * K4_TOK
        i0 = pltpu.async_copy(pos_hbm.at[pl.ds(t0, K4_TOK)], idx0_v, rsem)
        i1 = pltpu.async_copy(pos_hbm.at[pl.ds(S + t0, K4_TOK)], idx1_v, rsem)
        g0 = pltpu.async_copy(g1_hbm.at[pl.ds(t0, K4_TOK)], w0_v, rsem)
        g1c = pltpu.async_copy(g2_hbm.at[pl.ds(t0, K4_TOK)], w1_v, rsem)
        i0.wait()
        cp0 = pltpu.async_copy(so_hbm.at[idx0_v], rows_v, sem)
        i1.wait()
        cp1 = pltpu.async_copy(so_hbm.at[idx1_v], rows1_v, sem)
        g0.wait()
        g1c.wait()
        cp0.wait()
        cp1.wait()

        dnums = lax.GatherDimensionNumbers(
            offset_dims=(), collapsed_slice_dims=(0,), start_index_map=(0,))

        def per_token(t, _):
            base = (t // 16) * 16
            off = t - base

            def bcast(wv):
                return lax.gather(
                    wv[pl.ds(base, 16)], jnp.full((16, 1), off, jnp.int32),
                    dnums, slice_sizes=(1,),
                    mode=lax.GatherScatterMode.PROMISE_IN_BOUNDS)

            wv0 = bcast(w0_v)
            wv1 = bcast(w1_v)
            for c in range(D // 16):
                sl = pl.ds(c * 16, 16)
                rows_v[t, sl] = wv0 * rows_v[t, sl] + wv1 * rows1_v[t, sl]
            return 0

        lax.fori_loop(0, K4_TOK, per_token, 0)
        pltpu.sync_copy(rows_v, out_hbm.at[pl.ds(t0, K4_TOK)])

    return _combine


def kernel(x, Wr, W1, b1, W2, b2):
    orig_shape = x.shape
    x2 = x.reshape(-1, x.shape[-1])
    pos, g1, g2, be2 = _router(x2, Wr)
    xs = _make_dispatch()(x2, pos)
    so = _grouped_mlp(be2, xs, W1, b1, W2, b2)
    out = _make_combine()(so, pos, g1, g2)
    return out.reshape(orig_shape)


# speedup vs baseline: 1.1206x; 1.0275x over previous
import functools
import jax
import jax.numpy as jnp
from jax import lax
from jax.experimental import pallas as pl
from jax.experimental.pallas import tpu as pltpu
from jax.experimental.pallas import tpu_sc as plsc

S, D, H, E, TOPK = 2048, 768, 768, 8, 2
NSLOT = S * TOPK
BLK = 256
NB = NSLOT // BLK + (E - 1)
NPAD = NB * BLK
CHUNK = 128
NCHUNK = NSLOT // CHUNK

NW = 32
K2_SLOTS = NSLOT // NW
K4_TOK = S // NW


def _router_body(x_ref, wr_ref, pos_ref, g1_ref, g2_ref, be_ref):
    xb = x_ref[...]
    logits = jnp.dot(xb, wr_ref[...], preferred_element_type=jnp.float32)
    lt = jnp.transpose(logits)

    Linc = (lax.broadcasted_iota(jnp.int32, (E, E), 0)
            >= lax.broadcasted_iota(jnp.int32, (E, E), 1)).astype(jnp.float32)
    m1 = jnp.max(lt, axis=0, keepdims=True)
    is1 = (lt == m1).astype(jnp.float32)
    first1 = is1 * (jnp.dot(Linc, is1, preferred_element_type=jnp.float32) == 1.0)
    masked = jnp.where(first1 > 0, -jnp.inf, lt)
    m2 = jnp.max(masked, axis=0, keepdims=True)
    is2 = (masked == m2).astype(jnp.float32)
    first2 = is2 * (jnp.dot(Linc, is2, preferred_element_type=jnp.float32) == 1.0)

    g1 = jax.nn.sigmoid(m1 - m2)
    g1_ref[...] = g1[0]
    g2_ref[...] = 1.0 - g1[0]

    ohT = jnp.concatenate([first1, first2], axis=1)
    cnt = jnp.sum(ohT, axis=1, keepdims=True)
    pcnt = jnp.floor((cnt + (BLK - 1)) * (1.0 / BLK)) * BLK
    SL = (lax.broadcasted_iota(jnp.int32, (E, E), 0)
          > lax.broadcasted_iota(jnp.int32, (E, E), 1)).astype(jnp.float32)
    off = jnp.dot(SL, pcnt, preferred_element_type=jnp.float32)

    US = (lax.broadcasted_iota(jnp.int32, (CHUNK, CHUNK), 0)
          < lax.broadcasted_iota(jnp.int32, (CHUNK, CHUNK), 1)
          ).astype(jnp.float32)
    run = jnp.zeros((E, 1), jnp.float32)
    for c in range(NCHUNK):
        ohc = ohT[:, c * CHUNK:(c + 1) * CHUNK]
        within = jnp.dot(ohc, US, preferred_element_type=jnp.float32)
        posc = jnp.sum((within + run + off) * ohc, axis=0)
        pos_ref[pl.ds(c * CHUNK, CHUNK)] = posc.astype(jnp.int32)
        run = run + jnp.sum(ohc, axis=1, keepdims=True)

    bs = (lax.broadcasted_iota(jnp.int32, (E, NB), 1) * BLK).astype(jnp.float32)
    bem = jnp.sum((bs >= off).astype(jnp.float32), axis=0, keepdims=True) - 1.0
    nrb = jnp.sum(pcnt, axis=0, keepdims=True) * (1.0 / BLK)
    be_ref[...] = jnp.concatenate([bem, nrb], axis=1).astype(jnp.int32)


@jax.jit
def _router(x2, Wr):
    return pl.pallas_call(
        _router_body,
        grid=(1,),
        in_specs=[
            pl.BlockSpec((S, D), lambda i: (0, 0)),
            pl.BlockSpec((D, E), lambda i: (0, 0)),
        ],
        out_specs=[
            pl.BlockSpec((NSLOT,), lambda i: (0,)),
            pl.BlockSpec((S,), lambda i: (0,)),
            pl.BlockSpec((S,), lambda i: (0,)),
            pl.BlockSpec((1, NB + 1), lambda i: (0, 0)),
        ],
        out_shape=[
            jax.ShapeDtypeStruct((NSLOT,), jnp.int32),
            jax.ShapeDtypeStruct((S,), jnp.float32),
            jax.ShapeDtypeStruct((S,), jnp.float32),
            jax.ShapeDtypeStruct((1, NB + 1), jnp.int32),
        ],
    )(x2, Wr)


@functools.lru_cache(maxsize=None)
def _make_dispatch():
    mesh = plsc.VectorSubcoreMesh(core_axis_name="c", subcore_axis_name="s")

    NCH = 4
    CH = K2_SLOTS // NCH

    @functools.partial(
        pl.kernel,
        out_type=jax.ShapeDtypeStruct((NPAD, D), jnp.float32),
        mesh=mesh,
        scratch_types=[
            [pltpu.VMEM((CH,), jnp.int32) for _ in range(NCH)],
            [pltpu.VMEM((CH, D), jnp.float32) for _ in range(NCH)],
            pltpu.SemaphoreType.DMA,
            pltpu.SemaphoreType.DMA,
        ],
    )
    def _dispatch(x_hbm, pos_hbm, xs_hbm, idxs, rows, rsem, sem):
        wid = lax.axis_index("s") * 2 + lax.axis_index("c")
        base = wid * K2_SLOTS
        t0 = lax.rem(base, S)
        rcps = []
        for j in range(NCH):
            rcps.append(pltpu.async_copy(
                pos_hbm.at[pl.ds(base + j * CH, CH)], idxs[j], rsem))
            rcps.append(pltpu.async_copy(
                x_hbm.at[pl.ds(t0 + j * CH, CH)], rows[j], rsem))
        cps = []
        for j in range(NCH):
            rcps[2 * j].wait()
            rcps[2 * j + 1].wait()
            cps.append(pltpu.async_copy(rows[j], xs_hbm.at[idxs[j]], sem))
        for cp in cps:
            cp.wait()

    return _dispatch


def _gelu_exact(x):
    return 0.5 * x * (1.0 + lax.erf(x * jnp.float32(0.7071067811865476)))


def _mlp_body(be_ref, xs_ref, w1_ref, b1_ref, w2_ref, b2_ref, o_ref):
    i = pl.program_id(0)

    @pl.when(i < be_ref[0, NB])
    def _():
        e = be_ref[0, i]
        h = jnp.dot(xs_ref[...], w1_ref[0], preferred_element_type=jnp.float32)
        h = _gelu_exact(h + b1_ref[pl.ds(e, 1)])
        o = jnp.dot(h, w2_ref[0], preferred_element_type=jnp.float32)
        o_ref[...] = o + b2_ref[pl.ds(e, 1)]


@jax.jit
def _grouped_mlp(be, xs, W1, b1, W2, b2):
    grid_spec = pltpu.PrefetchScalarGridSpec(
        num_scalar_prefetch=1,
        grid=(NB,),
        in_specs=[
            pl.BlockSpec((BLK, D), lambda i, be: (i, 0)),
            pl.BlockSpec((1, D, H), lambda i, be: (be[0, i], 0, 0)),
            pl.BlockSpec((E, H), lambda i, be: (0, 0)),
            pl.BlockSpec((1, H, D), lambda i, be: (be[0, i], 0, 0)),
            pl.BlockSpec((E, D), lambda i, be: (0, 0)),
        ],
        out_specs=pl.BlockSpec((BLK, D), lambda i, be: (i, 0)),
    )
    return pl.pallas_call(
        _mlp_body,
        grid_spec=grid_spec,
        out_shape=jax.ShapeDtypeStruct((NPAD, D), jnp.float32),
        compiler_params=pltpu.CompilerParams(
            dimension_semantics=("arbitrary",),
        ),
    )(be, xs, W1, b1, W2, b2)


@functools.lru_cache(maxsize=None)
def _make_combine():
    mesh = plsc.VectorSubcoreMesh(core_axis_name="c", subcore_axis_name="s")

    HT = K4_TOK // 2

    @functools.partial(
        pl.kernel,
        out_type=jax.ShapeDtypeStruct((S, D), jnp.float32),
        mesh=mesh,
        scratch_types=[
            [pltpu.VMEM((HT,), jnp.int32) for _ in range(4)],
            pltpu.VMEM((K4_TOK,), jnp.float32),
            pltpu.VMEM((K4_TOK,), jnp.float32),
            [pltpu.VMEM((HT, D), jnp.float32) for _ in range(4)],
            pltpu.SemaphoreType.DMA,
            pltpu.SemaphoreType.DMA,
            pltpu.SemaphoreType.DMA,
        ],
    )
    def _combine(so_hbm, pos_hbm, g1_hbm, g2_hbm, out_hbm,
                 idxs, w0_v, w1_v, rows, rsem, sem, wsem):
        wid = lax.axis_index("s") * 2 + lax.axis_index("c")
        t0 = wid * K4_TOK
        ics = []
        for h in range(2):
            ics.append(pltpu.async_copy(
                pos_hbm.at[pl.ds(t0 + h * HT, HT)], idxs[2 * h], rsem))
            ics.append(pltpu.async_copy(
                pos_hbm.at[pl.ds(S + t0 + h * HT, HT)], idxs[2 * h + 1], rsem))
        g0 = pltpu.async_copy(g1_hbm.at[pl.ds(t0, K4_TOK)], w0_v, rsem)
        g1c = pltpu.async_copy(g2_hbm.at[pl.ds(t0, K4_TOK)], w1_v, rsem)
        cps = []
        for h in range(2):
            ics[2 * h].wait()
            ics[2 * h + 1].wait()
            cps.append(pltpu.async_copy(so_hbm.at[idxs[2 * h]], rows[2 * h], sem))
            cps.append(pltpu.async_copy(so_hbm.at[idxs[2 * h + 1]],
                                        rows[2 * h + 1], sem))
        g0.wait()
        g1c.wait()

        dnums = lax.GatherDimensionNumbers(
            offset_dims=(), collapsed_slice_dims=(0,), start_index_map=(0,))
        wcs = []
        for h in range(2):
            cps[2 * h].wait()
            cps[2 * h + 1].wait()
            r0, r1 = rows[2 * h], rows[2 * h + 1]

            def per_token(t, _, r0=r0, r1=r1, h=h):
                base = (t // 16) * 16
                off = t - base

                def bcast(wv):
                    return lax.gather(
                        wv[pl.ds(h * HT + base, 16)],
                        jnp.full((16, 1), off, jnp.int32),
                        dnums, slice_sizes=(1,),
                        mode=lax.GatherScatterMode.PROMISE_IN_BOUNDS)

                wv0 = bcast(w0_v)
                wv1 = bcast(w1_v)
                for c in range(D // 16):
                    sl = pl.ds(c * 16, 16)
                    r0[t, sl] = wv0 * r0[t, sl] + wv1 * r1[t, sl]
                return 0

            lax.fori_loop(0, HT, per_token, 0)
            wcs.append(pltpu.async_copy(
                r0, out_hbm.at[pl.ds(t0 + h * HT, HT)], wsem))
        for wc in wcs:
            wc.wait()

    return _combine


def kernel(x, Wr, W1, b1, W2, b2):
    orig_shape = x.shape
    x2 = x.reshape(-1, x.shape[-1])
    pos, g1, g2, be2 = _router(x2, Wr)
    xs = _make_dispatch()(x2, pos)
    so = _grouped_mlp(be2, xs, W1, b1, W2, b2)
    out = _make_combine()(so, pos, g1, g2)
    return out.reshape(orig_shape)
